# Initial kernel scaffold; baseline (speedup 1.0000x reference)
import jax
import jax.numpy as jnp
from jax.experimental import pallas as pl


def kernel(x, adj_t, node_type, edge_type, gamma1, beta1, W_conv, gamma2, beta2, W_mlp, b_mlp):
    raise NotImplementedError("write your pallas kernel here")


# speedup vs baseline: 12.0401x; 12.0401x over previous
import functools

import jax
import jax.numpy as jnp
from jax import lax
from jax.experimental import pallas as pl
from jax.experimental.pallas import tpu as pltpu
from jax.experimental.pallas import tpu_sc as plsc

EPS = 1e-5

_NC = 2
_NS = 16
_NW = _NC * _NS
_CHUNK = 128
_NBUF = 2


def _ln_act(xb, oh, g_ref, b_ref):
    mu = jnp.mean(xb, axis=-1, keepdims=True)
    var = jnp.mean((xb - mu) ** 2, axis=-1, keepdims=True)
    h = (xb - mu) * lax.rsqrt(var + EPS)
    g = jnp.dot(oh, g_ref[...], preferred_element_type=jnp.float32)
    b = jnp.dot(oh, b_ref[...], preferred_element_type=jnp.float32)
    return jax.nn.relu(h * g + b)


def _stage_a_body(x_ref, oh_ref, g1_ref, b1_ref, w_ref, out_ref):
    y = _ln_act(x_ref[...], oh_ref[...], g1_ref, b1_ref)
    for r in range(w_ref.shape[0]):
        out_ref[r] = jnp.dot(y, w_ref[r], preferred_element_type=jnp.float32,
                             precision=lax.Precision.HIGHEST)


def _stage_c_body(x_ref, a_ref, oh_ref, g2_ref, b2_ref, wm_ref, bm_ref, out_ref):
    x2 = x_ref[...] + a_ref[0] + a_ref[1]
    y = _ln_act(x2, oh_ref[...], g2_ref, b2_ref)
    oh = oh_ref[...]
    acc = x2 + jnp.dot(oh, bm_ref[...], preferred_element_type=jnp.float32)
    for t in range(wm_ref.shape[0]):
        zt = jnp.dot(y, wm_ref[t], preferred_element_type=jnp.float32,
                     precision=lax.Precision.HIGHEST)
        acc = acc + oh[:, t][:, None] * zt
    out_ref[...] = acc


def _make_sc_scatter(n_rows_table, npad, ch, d):
    rows_per = npad // _NS
    mesh = plsc.VectorSubcoreMesh(core_axis_name="c", subcore_axis_name="s")

    @functools.partial(
        pl.kernel,
        mesh=mesh,
        out_type=jax.ShapeDtypeStruct((_NC, npad, d), jnp.float32),
        scratch_types=[
            pltpu.VMEM((2, 2, _CHUNK), jnp.int32),
            pltpu.VMEM((2, _CHUNK, d), jnp.float32),
            pltpu.VMEM_SHARED((npad, d), jnp.float32),
            pltpu.SemaphoreType.DMA,
            pltpu.SemaphoreType.DMA,
            pltpu.SemaphoreType.DMA,
            pltpu.SemaphoreType.DMA,
        ],
    )
    def sc_scatter(hs, idx4, zeros, out, ib, rows, acc, i0, i1, g0, g1):
        c = lax.axis_index("c")
        s = lax.axis_index("s")
        w = c * _NS + s
        isems = (i0, i1)
        gsems = (g0, g1)
        pltpu.sync_copy(zeros, acc.at[pl.ds(s * rows_per, rows_per)])
        plsc.subcore_barrier()

        pltpu.async_copy(idx4.at[w, 0], ib.at[0], isems[0])
        pltpu.async_copy(idx4.at[w, 1], ib.at[1], isems[1])
        pltpu.make_async_copy(idx4.at[w, 0], ib.at[0], isems[0]).wait()
        pltpu.async_copy(hs.at[ib.at[0, 0]], rows.at[0], gsems[0])

        def body(j):
            for b in range(2):
                jj = j + b
                @pl.when(jj + 1 < ch)
                def _():
                    pltpu.make_async_copy(idx4.at[w, jj + 1], ib.at[1 - b],
                                          isems[1 - b]).wait()
                    pltpu.async_copy(hs.at[ib.at[1 - b, 0]], rows.at[1 - b],
                                     gsems[1 - b])

                pltpu.make_async_copy(hs.at[ib.at[b, 0]], rows.at[b],
                                      gsems[b]).wait()
                pltpu.sync_copy(rows.at[b], acc.at[ib.at[b, 1]], add=True)

                @pl.when(jj + 2 < ch)
                def _():
                    pltpu.async_copy(idx4.at[w, jj + 2], ib.at[b], isems[b])

        pl.loop(0, ch, step=2)(body)
        plsc.subcore_barrier()
        pltpu.sync_copy(acc.at[pl.ds(s * rows_per, rows_per)],
                        out.at[c, pl.ds(s * rows_per, rows_per)])

    return sc_scatter


def kernel(x, adj_t, node_type, edge_type, gamma1, beta1, W_conv, gamma2,
           beta2, W_mlp, b_mlp):
    n, d = x.shape
    r = W_conv.shape[0]
    t = W_mlp.shape[0]
    e = edge_type.shape[0]

    nt = node_type.astype(jnp.int32)
    onehot = jax.nn.one_hot(nt, t, dtype=jnp.float32)

    bn = 1000
    grid_a = n // bn
    hs = pl.pallas_call(
        _stage_a_body,
        grid=(grid_a,),
        in_specs=[
            pl.BlockSpec((bn, d), lambda i: (i, 0)),
            pl.BlockSpec((bn, t), lambda i: (i, 0)),
            pl.BlockSpec((t, d), lambda i: (0, 0)),
            pl.BlockSpec((t, d), lambda i: (0, 0)),
            pl.BlockSpec((r, d, d), lambda i: (0, 0, 0)),
        ],
        out_specs=pl.BlockSpec((r, bn, d), lambda i: (0, i, 0)),
        out_shape=jax.ShapeDtypeStruct((r, n, d), jnp.float32),
    )(x, onehot, gamma1, beta1, W_conv)
    hs_flat = hs.reshape(r * n, d)

    src = adj_t[0].astype(jnp.int32)
    dst = adj_t[1].astype(jnp.int32)
    et = edge_type.astype(jnp.int32)
    gidx = et * n + src

    ew = -(-e // (_NW * _CHUNK * _NBUF)) * (_CHUNK * _NBUF)
    ch = ew // _CHUNK
    e_pad = _NW * ew
    npad = -(-(n + 1) // (_NS * 8)) * (_NS * 8)
    assert (npad // _NS) % 8 == 0

    pad = e_pad - e
    gidx_p = jnp.concatenate([gidx, jnp.zeros((pad,), jnp.int32)])
    didx_p = jnp.concatenate([dst, jnp.full((pad,), n, jnp.int32)])
    gidx3 = gidx_p.reshape(_NW, ch, _CHUNK)
    didx3 = didx_p.reshape(_NW, ch, _CHUNK)
    idx4 = jnp.stack([gidx3, didx3], axis=2)
    zeros = jnp.zeros((npad // _NS, d), jnp.float32)

    agg2 = _make_sc_scatter(r * n, npad, ch, d)(hs_flat, idx4, zeros)

    out = pl.pallas_call(
        _stage_c_body,
        grid=(grid_a,),
        in_specs=[
            pl.BlockSpec((bn, d), lambda i: (i, 0)),
            pl.BlockSpec((_NC, bn, d), lambda i: (0, i, 0)),
            pl.BlockSpec((bn, t), lambda i: (i, 0)),
            pl.BlockSpec((t, d), lambda i: (0, 0)),
            pl.BlockSpec((t, d), lambda i: (0, 0)),
            pl.BlockSpec((t, d, d), lambda i: (0, 0, 0)),
            pl.BlockSpec((t, d), lambda i: (0, 0)),
        ],
        out_specs=pl.BlockSpec((bn, d), lambda i: (i, 0)),
        out_shape=jax.ShapeDtypeStruct((n, d), jnp.float32),
    )(x, agg2, onehot, gamma2, beta2, W_mlp, b_mlp)
    return out


# speedup vs baseline: 15.1025x; 1.2543x over previous
import functools

import jax
import jax.numpy as jnp
from jax import lax
from jax.experimental import pallas as pl
from jax.experimental.pallas import tpu as pltpu
from jax.experimental.pallas import tpu_sc as plsc

EPS = 1e-5

_NC = 2
_NS = 16
_NW = _NC * _NS
_CHUNK = 128
_NBUF = 2


def _ln_act(xb, oh, g_ref, b_ref):
    mu = jnp.mean(xb, axis=-1, keepdims=True)
    var = jnp.mean((xb - mu) ** 2, axis=-1, keepdims=True)
    h = (xb - mu) * lax.rsqrt(var + EPS)
    g = jnp.dot(oh, g_ref[...], preferred_element_type=jnp.float32)
    b = jnp.dot(oh, b_ref[...], preferred_element_type=jnp.float32)
    return jax.nn.relu(h * g + b)


def _stage_a_body(x_ref, oh_ref, g1_ref, b1_ref, w_ref, out_ref):
    y = _ln_act(x_ref[...], oh_ref[...], g1_ref, b1_ref)
    for r in range(w_ref.shape[0]):
        out_ref[r] = jnp.dot(y, w_ref[r], preferred_element_type=jnp.float32,
                             precision=lax.Precision.HIGHEST)


def _stage_c_body(x_ref, a_ref, oh_ref, g2_ref, b2_ref, wm_ref, bm_ref, out_ref):
    x2 = x_ref[...] + a_ref[0] + a_ref[1]
    y = _ln_act(x2, oh_ref[...], g2_ref, b2_ref)
    oh = oh_ref[...]
    acc = x2 + jnp.dot(oh, bm_ref[...], preferred_element_type=jnp.float32)
    for t in range(wm_ref.shape[0]):
        zt = jnp.dot(y, wm_ref[t], preferred_element_type=jnp.float32,
                     precision=lax.Precision.HIGHEST)
        acc = acc + oh[:, t][:, None] * zt
    out_ref[...] = acc


def _make_sc_scatter(n_rows_table, npad, ch0, ch1, d):
    rows_per = npad // _NS
    mesh = plsc.VectorSubcoreMesh(core_axis_name="c", subcore_axis_name="s")

    @functools.partial(
        pl.kernel,
        mesh=mesh,
        out_type=jax.ShapeDtypeStruct((_NC, npad, d), jnp.float32),
        scratch_types=[
            pltpu.VMEM((2, 2, _CHUNK), jnp.int32),
            pltpu.VMEM((2, _CHUNK, d), jnp.float32),
            pltpu.VMEM_SHARED((npad, d), jnp.float32),
            pltpu.SemaphoreType.DMA,
            pltpu.SemaphoreType.DMA,
            pltpu.SemaphoreType.DMA,
            pltpu.SemaphoreType.DMA,
        ],
    )
    def sc_scatter(hs, idx4, zeros, out, ib, rows, acc, i0, i1, g0, g1):
        c = lax.axis_index("c")
        s = lax.axis_index("s")
        w = c * _NS + s
        nch = lax.select(c == 0, jnp.int32(ch0), jnp.int32(ch1))
        isems = (i0, i1)
        gsems = (g0, g1)
        pltpu.sync_copy(zeros, acc.at[pl.ds(s * rows_per, rows_per)])
        plsc.subcore_barrier()

        pltpu.async_copy(idx4.at[w, 0], ib.at[0], isems[0])
        pltpu.async_copy(idx4.at[w, 1], ib.at[1], isems[1])
        pltpu.make_async_copy(idx4.at[w, 0], ib.at[0], isems[0]).wait()
        pltpu.async_copy(hs.at[ib.at[0, 0]], rows.at[0], gsems[0])

        def body(j):
            for b in range(2):
                jj = j + b
                @pl.when(jj + 1 < nch)
                def _():
                    pltpu.make_async_copy(idx4.at[w, jj + 1], ib.at[1 - b],
                                          isems[1 - b]).wait()
                    pltpu.async_copy(hs.at[ib.at[1 - b, 0]], rows.at[1 - b],
                                     gsems[1 - b])

                pltpu.make_async_copy(hs.at[ib.at[b, 0]], rows.at[b],
                                      gsems[b]).wait()
                pltpu.sync_copy(rows.at[b], acc.at[ib.at[b, 1]], add=True)

                @pl.when(jj + 2 < nch)
                def _():
                    pltpu.async_copy(idx4.at[w, jj + 2], ib.at[b], isems[b])

        pl.loop(0, nch, step=2)(body)
        plsc.subcore_barrier()
        pltpu.sync_copy(acc.at[pl.ds(s * rows_per, rows_per)],
                        out.at[c, pl.ds(s * rows_per, rows_per)])

    return sc_scatter


def kernel(x, adj_t, node_type, edge_type, gamma1, beta1, W_conv, gamma2,
           beta2, W_mlp, b_mlp):
    n, d = x.shape
    r = W_conv.shape[0]
    t = W_mlp.shape[0]
    e = edge_type.shape[0]

    nt = node_type.astype(jnp.int32)
    onehot = jax.nn.one_hot(nt, t, dtype=jnp.float32)

    bn = 1000
    grid_a = n // bn
    hs = pl.pallas_call(
        _stage_a_body,
        grid=(grid_a,),
        in_specs=[
            pl.BlockSpec((bn, d), lambda i: (i, 0)),
            pl.BlockSpec((bn, t), lambda i: (i, 0)),
            pl.BlockSpec((t, d), lambda i: (0, 0)),
            pl.BlockSpec((t, d), lambda i: (0, 0)),
            pl.BlockSpec((r, d, d), lambda i: (0, 0, 0)),
        ],
        out_specs=pl.BlockSpec((r, bn, d), lambda i: (0, i, 0)),
        out_shape=jax.ShapeDtypeStruct((r, n, d), jnp.float32),
    )(x, onehot, gamma1, beta1, W_conv)
    hs_flat = hs.reshape(r * n, d)

    src = adj_t[0].astype(jnp.int32)
    dst = adj_t[1].astype(jnp.int32)
    et = edge_type.astype(jnp.int32)
    gidx = et * n + src

    ctot = -(-e // (_NS * _CHUNK))
    ch0 = min(ctot, max(2, 2 * round(0.73 * ctot / 2)))
    ch1 = max(2, -(-(ctot - ch0) // 2) * 2)
    e0 = _NS * ch0 * _CHUNK
    cap1 = _NS * ch1 * _CHUNK
    npad = -(-(n + 1) // (_NS * 8)) * (_NS * 8)
    assert (npad // _NS) % 8 == 0

    pad = e0 + cap1 - e
    gidx_p = jnp.concatenate([gidx, jnp.zeros((pad,), jnp.int32)])
    didx_p = jnp.concatenate([dst, jnp.full((pad,), n, jnp.int32)])
    chm = max(ch0, ch1)

    def _core_idx(gi, di, chc):
        g3 = gi.reshape(_NS, chc, _CHUNK)
        d3 = di.reshape(_NS, chc, _CHUNK)
        i4 = jnp.stack([g3, d3], axis=2)
        return jnp.pad(i4, ((0, 0), (0, chm - chc), (0, 0), (0, 0)))

    idx4 = jnp.concatenate([
        _core_idx(gidx_p[:e0], didx_p[:e0], ch0),
        _core_idx(gidx_p[e0:], didx_p[e0:], ch1),
    ])
    zeros = jnp.zeros((npad // _NS, d), jnp.float32)

    agg2 = _make_sc_scatter(r * n, npad, ch0, ch1, d)(hs_flat, idx4, zeros)

    out = pl.pallas_call(
        _stage_c_body,
        grid=(grid_a,),
        in_specs=[
            pl.BlockSpec((bn, d), lambda i: (i, 0)),
            pl.BlockSpec((_NC, bn, d), lambda i: (0, i, 0)),
            pl.BlockSpec((bn, t), lambda i: (i, 0)),
            pl.BlockSpec((t, d), lambda i: (0, 0)),
            pl.BlockSpec((t, d), lambda i: (0, 0)),
            pl.BlockSpec((t, d, d), lambda i: (0, 0, 0)),
            pl.BlockSpec((t, d), lambda i: (0, 0)),
        ],
        out_specs=pl.BlockSpec((bn, d), lambda i: (i, 0)),
        out_shape=jax.ShapeDtypeStruct((n, d), jnp.float32),
    )(x, agg2, onehot, gamma2, beta2, W_mlp, b_mlp)
    return out


# speedup vs baseline: 19.9249x; 1.3193x over previous
import functools

import jax
import jax.numpy as jnp
from jax import lax
from jax.experimental import pallas as pl
from jax.experimental.pallas import tpu as pltpu
from jax.experimental.pallas import tpu_sc as plsc

EPS = 1e-5

_NC = 2
_NS = 16
_CHUNK = 128


def _ln_act(xb, oh, g_ref, b_ref):
    mu = jnp.mean(xb, axis=-1, keepdims=True)
    var = jnp.mean((xb - mu) ** 2, axis=-1, keepdims=True)
    h = (xb - mu) * lax.rsqrt(var + EPS)
    g = jnp.dot(oh, g_ref[...], preferred_element_type=jnp.float32)
    b = jnp.dot(oh, b_ref[...], preferred_element_type=jnp.float32)
    return jax.nn.relu(h * g + b)


def _stage_a_body(x_ref, oh_ref, g1_ref, b1_ref, w_ref, out_ref):
    d = x_ref.shape[1]
    r = w_ref.shape[1] // d
    y = _ln_act(x_ref[...], oh_ref[...], g1_ref, b1_ref)
    hs = jnp.dot(y, w_ref[...], preferred_element_type=jnp.float32)
    for i in range(r):
        out_ref[i] = hs[:, i * d:(i + 1) * d]


def _stage_c_body(x_ref, a_ref, oh_ref, g2_ref, b2_ref, wm_ref, bm_ref, out_ref):
    d = x_ref.shape[1]
    t = wm_ref.shape[1] // d
    x2 = x_ref[...] + a_ref[0] + a_ref[1]
    oh = oh_ref[...]
    y = _ln_act(x2, oh, g2_ref, b2_ref)
    z = jnp.dot(y, wm_ref[...], preferred_element_type=jnp.float32)
    acc = x2 + jnp.dot(oh, bm_ref[...], preferred_element_type=jnp.float32)
    for i in range(t):
        acc = acc + oh[:, i][:, None] * z[:, i * d:(i + 1) * d]
    out_ref[...] = acc


def _make_sc_scatter(npad, ch0, ch1, d):
    rows_per = npad // _NS
    mesh = plsc.VectorSubcoreMesh(core_axis_name="c", subcore_axis_name="s")

    @functools.partial(
        pl.kernel,
        mesh=mesh,
        out_type=jax.ShapeDtypeStruct((_NC, npad, d), jnp.float32),
        scratch_types=[
            pltpu.VMEM((2, _CHUNK), jnp.int32),
            pltpu.VMEM((2, _CHUNK), jnp.int32),
            pltpu.VMEM((2, _CHUNK, d), jnp.float32),
            pltpu.VMEM_SHARED((npad, d), jnp.float32),
            pltpu.SemaphoreType.DMA,
            pltpu.SemaphoreType.DMA,
            pltpu.SemaphoreType.DMA,
            pltpu.SemaphoreType.DMA,
        ],
    )
    def sc_scatter(hs, gidx, didx, zeros, out, gv, dv, rows, acc,
                   i0, i1, g0, g1):
        c = lax.axis_index("c")
        s = lax.axis_index("s")
        nch = lax.select(c == 0, jnp.int32(ch0), jnp.int32(ch1))
        base = lax.select(c == 0, s * ch0, _NS * ch0 + s * ch1)
        isems = (i0, i1)
        gsems = (g0, g1)
        pltpu.sync_copy(zeros, acc.at[pl.ds(s * rows_per, rows_per)])
        plsc.subcore_barrier()

        def load_idx(j, b):
            pltpu.async_copy(gidx.at[base + j], gv.at[b], isems[b])
            pltpu.async_copy(didx.at[base + j], dv.at[b], isems[b])

        def wait_idx(j, b):
            pltpu.make_async_copy(gidx.at[base + j], gv.at[b], isems[b]).wait()
            pltpu.make_async_copy(didx.at[base + j], dv.at[b], isems[b]).wait()

        load_idx(0, 0)
        load_idx(1, 1)
        wait_idx(0, 0)
        pltpu.async_copy(hs.at[gv.at[0]], rows.at[0], gsems[0])

        def body(j):
            for b in range(2):
                jj = j + b
                @pl.when(jj + 1 < nch)
                def _():
                    wait_idx(jj + 1, 1 - b)
                    pltpu.async_copy(hs.at[gv.at[1 - b]], rows.at[1 - b],
                                     gsems[1 - b])

                pltpu.make_async_copy(hs.at[gv.at[b]], rows.at[b],
                                      gsems[b]).wait()
                pltpu.sync_copy(rows.at[b], acc.at[dv.at[b]], add=True)

                @pl.when(jj + 2 < nch)
                def _():
                    load_idx(jj + 2, b)

        pl.loop(0, nch, step=2)(body)
        plsc.subcore_barrier()
        pltpu.sync_copy(acc.at[pl.ds(s * rows_per, rows_per)],
                        out.at[c, pl.ds(s * rows_per, rows_per)])

    return sc_scatter


def kernel(x, adj_t, node_type, edge_type, gamma1, beta1, W_conv, gamma2,
           beta2, W_mlp, b_mlp):
    n, d = x.shape
    r = W_conv.shape[0]
    t = W_mlp.shape[0]
    e = edge_type.shape[0]

    nt = node_type.astype(jnp.int32)
    onehot = jax.nn.one_hot(nt, t, dtype=jnp.float32)

    bn = 1000
    grid_a = n // bn
    hs = pl.pallas_call(
        _stage_a_body,
        grid=(grid_a,),
        in_specs=[
            pl.BlockSpec((bn, d), lambda i: (i, 0)),
            pl.BlockSpec((bn, t), lambda i: (i, 0)),
            pl.BlockSpec((t, d), lambda i: (0, 0)),
            pl.BlockSpec((t, d), lambda i: (0, 0)),
            pl.BlockSpec((d, r * d), lambda i: (0, 0)),
        ],
        out_specs=pl.BlockSpec((r, bn, d), lambda i: (0, i, 0)),
        out_shape=jax.ShapeDtypeStruct((r, n, d), jnp.float32),
    )(x, onehot, gamma1, beta1,
      jnp.moveaxis(W_conv, 0, 1).reshape(d, r * d))
    hs_flat = hs.reshape(r * n, d)

    src = adj_t[0].astype(jnp.int32)
    dst = adj_t[1].astype(jnp.int32)
    et = edge_type.astype(jnp.int32)
    gidx = et * n + src

    ctot = -(-e // (_NS * _CHUNK))
    ch0 = min(ctot, max(2, 2 * round(0.86 * ctot / 2)))
    ch1 = max(2, -(-(ctot - ch0) // 2) * 2)
    e0 = _NS * ch0 * _CHUNK
    cap = e0 + _NS * ch1 * _CHUNK
    npad = -(-(n + 1) // (_NS * 8)) * (_NS * 8)
    assert (npad // _NS) % 8 == 0

    pad = cap - e
    gidx2 = jnp.concatenate([gidx, jnp.zeros((pad,), jnp.int32)])
    didx2 = jnp.concatenate([dst, jnp.full((pad,), n, jnp.int32)])
    gidx2 = gidx2.reshape(-1, _CHUNK)
    didx2 = didx2.reshape(-1, _CHUNK)
    zeros = jnp.zeros((npad // _NS, d), jnp.float32)

    agg2 = _make_sc_scatter(npad, ch0, ch1, d)(hs_flat, gidx2, didx2, zeros)

    out = pl.pallas_call(
        _stage_c_body,
        grid=(grid_a,),
        in_specs=[
            pl.BlockSpec((bn, d), lambda i: (i, 0)),
            pl.BlockSpec((_NC, bn, d), lambda i: (0, i, 0)),
            pl.BlockSpec((bn, t), lambda i: (i, 0)),
            pl.BlockSpec((t, d), lambda i: (0, 0)),
            pl.BlockSpec((t, d), lambda i: (0, 0)),
            pl.BlockSpec((d, t * d), lambda i: (0, 0)),
            pl.BlockSpec((t, d), lambda i: (0, 0)),
        ],
        out_specs=pl.BlockSpec((bn, d), lambda i: (i, 0)),
        out_shape=jax.ShapeDtypeStruct((n, d), jnp.float32),
    )(x, agg2, onehot, gamma2, beta2,
      jnp.moveaxis(W_mlp, 0, 1).reshape(d, t * d), b_mlp)
    return out
